# Initial kernel scaffold; baseline (speedup 1.0000x reference)
import jax
import jax.numpy as jnp
from jax.experimental import pallas as pl


def kernel(x, h, emb_table):
    raise NotImplementedError("write your pallas kernel here")


# speedup vs baseline: 4.0889x; 4.0889x over previous
import functools

import jax
import jax.numpy as jnp
from jax import lax
from jax.experimental import pallas as pl
from jax.experimental.pallas import tpu as pltpu
from jax.experimental.pallas import tpu_sc as plsc

VSIZE = 100000
WORD_DIM = 64
BATCH = 4096
HIST = 50

NC, NS = 2, 16
NW = NC * NS
B = BATCH * HIST
BPW = B // NW
CHUNK = 128
NCHUNK = BPW // CHUNK

_mesh = plsc.VectorSubcoreMesh(
    core_axis_name="c", subcore_axis_name="s", num_cores=NC, num_subcores=NS
)


@functools.partial(
    pl.kernel,
    out_type=jax.ShapeDtypeStruct((B, WORD_DIM), jnp.float32),
    mesh=_mesh,
    scratch_types=[
        pltpu.VMEM((BPW,), jnp.int32),
        pltpu.VMEM((CHUNK, WORD_DIM), jnp.float32),
        pltpu.SemaphoreType.DMA,
    ],
    compiler_params=pltpu.CompilerParams(use_tc_tiling_on_sc=False),
)
def _emb_lookup(x_hbm, table_hbm, out_hbm, idx_v, rows_v, sem):
    wid = lax.axis_index("s") * NC + lax.axis_index("c")
    base = wid * BPW
    pltpu.sync_copy(x_hbm.at[pl.ds(base, BPW)], idx_v)

    def body(i, carry):
        idx_slice = idx_v.at[pl.ds(i * CHUNK, CHUNK)]
        pltpu.async_copy(table_hbm.at[idx_slice], rows_v, sem).wait()
        pltpu.sync_copy(rows_v, out_hbm.at[pl.ds(base + i * CHUNK, CHUNK)])
        return carry

    lax.fori_loop(0, NCHUNK, body, 0)


def kernel(x, h, emb_table):
    del h
    flat = _emb_lookup(x.reshape(-1), emb_table)
    return flat.reshape(BATCH, HIST, WORD_DIM)


# speedup vs baseline: 4.6431x; 1.1355x over previous
import functools

import jax
import jax.numpy as jnp
from jax import lax
from jax.experimental import pallas as pl
from jax.experimental.pallas import tpu as pltpu
from jax.experimental.pallas import tpu_sc as plsc

VSIZE = 100000
WORD_DIM = 64
BATCH = 4096
HIST = 50

NC, NS = 2, 16
NW = NC * NS
B = BATCH * HIST
BPW = B // NW
GCHUNK = 128
CHUNK = 256
GPB = CHUNK // GCHUNK
NBUF = 5
NCHUNK = BPW // CHUNK

_mesh = plsc.VectorSubcoreMesh(
    core_axis_name="c", subcore_axis_name="s", num_cores=NC, num_subcores=NS
)

_scratch = [
    pltpu.VMEM((BPW,), jnp.int32),
    pltpu.VMEM((NBUF, CHUNK, WORD_DIM), jnp.float32),
] + [pltpu.SemaphoreType.DMA] * (2 * NBUF)


@functools.partial(
    pl.kernel,
    out_type=jax.ShapeDtypeStruct((B, WORD_DIM), jnp.float32),
    mesh=_mesh,
    scratch_types=_scratch,
    compiler_params=pltpu.CompilerParams(use_tc_tiling_on_sc=False),
)
def _emb_lookup(x_hbm, table_hbm, out_hbm, idx_v, rows_v, *sems):
    sem_g = sems[:NBUF]
    sem_w = sems[NBUF:]
    wid = lax.axis_index("s") * NC + lax.axis_index("c")
    base = wid * BPW
    pltpu.sync_copy(x_hbm.at[pl.ds(base, BPW)], idx_v)

    def fire_gathers(i):
        b = i % NBUF
        descs = []
        for j in range(GPB):
            off = i * CHUNK + j * GCHUNK
            descs.append(
                pltpu.async_copy(
                    table_hbm.at[idx_v.at[pl.ds(off, GCHUNK)]],
                    rows_v.at[b, pl.ds(j * GCHUNK, GCHUNK)],
                    sem_g[b],
                )
            )
        return descs

    g_descs = {}
    w_descs = {}
    for i in range(NBUF):
        g_descs[i] = fire_gathers(i)

    for i in range(NCHUNK):
        b = i % NBUF
        for d in g_descs.pop(i):
            d.wait()
        w_descs[i] = pltpu.async_copy(
            rows_v.at[b], out_hbm.at[pl.ds(base + i * CHUNK, CHUNK)], sem_w[b]
        )
        p = i - 1
        if p >= 0:
            w_descs.pop(p).wait()
            if p + NBUF < NCHUNK:
                g_descs[p + NBUF] = fire_gathers(p + NBUF)
    w_descs.pop(NCHUNK - 1).wait()


def kernel(x, h, emb_table):
    del h
    flat = _emb_lookup(x.reshape(-1), emb_table)
    return flat.reshape(BATCH, HIST, WORD_DIM)


# speedup vs baseline: 17.0513x; 3.6724x over previous
import functools

import jax
import jax.numpy as jnp
from jax import lax
from jax.experimental import pallas as pl
from jax.experimental.pallas import tpu as pltpu
from jax.experimental.pallas import tpu_sc as plsc

VSIZE = 100000
WORD_DIM = 64
BATCH = 4096
HIST = 50

NC, NS = 2, 16
NW = NC * NS
FPW = WORD_DIM // NW
NPAIR = HIST // 2

_mesh = plsc.VectorSubcoreMesh(
    core_axis_name="c", subcore_axis_name="s", num_cores=NC, num_subcores=NS
)

_scratch = [
    pltpu.VMEM_SHARED((HIST * BATCH,), jnp.int32),
    pltpu.VMEM((VSIZE,), jnp.float32),
    pltpu.VMEM((BATCH,), jnp.int32),
    pltpu.VMEM((BATCH,), jnp.int32),
    pltpu.VMEM((BATCH,), jnp.float32),
    pltpu.VMEM((BATCH,), jnp.float32),
] + [pltpu.SemaphoreType.DMA] * 5


@functools.partial(
    pl.kernel,
    out_type=jax.ShapeDtypeStruct((HIST, WORD_DIM, BATCH), jnp.float32),
    mesh=_mesh,
    scratch_types=_scratch,
    compiler_params=pltpu.CompilerParams(
        use_tc_tiling_on_sc=True, needs_layout_passes=False
    ),
)
def _emb_lookup_t(xt_hbm, tt_hbm, out_hbm, xsh, trow, ib0, ib1, ob0, ob1,
                  sem_t, sem_i0, sem_i1, sem_o0, sem_o1):
    wid = lax.axis_index("s") * NC + lax.axis_index("c")

    t_desc0 = pltpu.async_copy(tt_hbm.at[wid], trow, sem_t)

    @pl.when(lax.axis_index("s") == 0)
    def _stage_idx():
        pltpu.sync_copy(xt_hbm, xsh)

    plsc.subcore_barrier()

    def gather(ib, ob):
        @plsc.parallel_loop(0, BATCH // 16, unroll=16)
        def gbody(j):
            off = j * 16
            iv = ib[pl.ds(off, 16)]
            ob[pl.ds(off, 16)] = plsc.load_gather(trow, [iv])

    def fire_idx(h, ib, sem):
        pltpu.async_copy(xsh.at[pl.ds(h * BATCH, BATCH)], ib, sem)

    def wait_idx(ib, sem):
        pltpu.make_async_copy(xsh.at[pl.ds(0, BATCH)], ib, sem).wait()

    def wait_out(ob, sem, h, d):
        pltpu.make_async_copy(ob, out_hbm.at[h, d], sem).wait()

    for f in range(FPW):
        d = wid + NW * f
        t_desc = t_desc0 if f == 0 else pltpu.async_copy(
            tt_hbm.at[d], trow, sem_t)
        fire_idx(0, ib0, sem_i0)
        fire_idx(1, ib1, sem_i1)
        t_desc.wait()

        wait_idx(ib0, sem_i0)
        gather(ib0, ob0)
        pltpu.async_copy(ob0, out_hbm.at[0, d], sem_o0)
        fire_idx(2, ib0, sem_i0)
        wait_idx(ib1, sem_i1)
        gather(ib1, ob1)
        pltpu.async_copy(ob1, out_hbm.at[1, d], sem_o1)
        fire_idx(3, ib1, sem_i1)

        def body(g, c):
            h = 2 * g
            wait_idx(ib0, sem_i0)
            wait_out(ob0, sem_o0, h - 2, d)
            gather(ib0, ob0)
            pltpu.async_copy(ob0, out_hbm.at[h, d], sem_o0)
            fire_idx(jnp.minimum(h + 2, HIST - 1), ib0, sem_i0)
            wait_idx(ib1, sem_i1)
            wait_out(ob1, sem_o1, h - 1, d)
            gather(ib1, ob1)
            pltpu.async_copy(ob1, out_hbm.at[h + 1, d], sem_o1)
            fire_idx(jnp.minimum(h + 3, HIST - 1), ib1, sem_i1)
            return c

        lax.fori_loop(1, HIST // 2, body, 0)

        wait_out(ob0, sem_o0, HIST - 2, d)
        wait_out(ob1, sem_o1, HIST - 1, d)
        wait_idx(ib0, sem_i0)
        wait_idx(ib1, sem_i1)


def kernel(x, h, emb_table):
    del h
    out = _emb_lookup_t(x.T.reshape(-1), emb_table.T)
    return out.transpose(2, 0, 1)
